# Initial kernel scaffold; baseline (speedup 1.0000x reference)
import jax
import jax.numpy as jnp
from jax.experimental import pallas as pl


def kernel(user_item_edge_index, user_test_edge_index, user_tag_edge_index, user_item_table, user_test_table, user_tag_table):
    raise NotImplementedError("write your pallas kernel here")


# speedup vs baseline: 4.0213x; 4.0213x over previous
import functools

import jax
import jax.numpy as jnp
from jax import lax
from jax.experimental import pallas as pl
from jax.experimental.pallas import tpu as pltpu
from jax.experimental.pallas import tpu_sc as plsc

N = 10000
D = 128
E = 320000
G = 3
L = 3
ALPHA = 0.25
LANES = 16
CHUNK = 128


def _zeros16():
    return jnp.zeros((LANES,), jnp.float32)


def _rsqrt16(d):
    i = lax.bitcast_convert_type(d, jnp.int32)
    i = jnp.full((LANES,), 0x5F3759DF, jnp.int32) - lax.shift_right_arithmetic(
        i, jnp.full((LANES,), 1, jnp.int32))
    r = lax.bitcast_convert_type(i, jnp.float32)
    for _ in range(3):
        r = r * (1.5 - 0.5 * d * r * r)
    return r


def _build(nc, ns):
    T = nc * ns
    C = -(-E // (T * CHUNK))
    EP = T * C * CHUNK
    NP = ((N + T * LANES - 1) // (T * LANES)) * (T * LANES)
    RPT = NP // T
    RPS = NP // ns
    mesh = plsc.VectorSubcoreMesh(core_axis_name="c", subcore_axis_name="s")

    def _wid():
        return lax.axis_index("s") * nc + lax.axis_index("c")

    @functools.partial(
        pl.kernel,
        out_type=jax.ShapeDtypeStruct((G, nc, NP), jnp.float32),
        mesh=mesh,
        compiler_params=pltpu.CompilerParams(use_tc_tiling_on_sc=False),
        scratch_types=[
            pltpu.VMEM((C, CHUNK), jnp.int32),
            pltpu.VMEM((CHUNK,), jnp.float32),
            pltpu.VMEM((RPS,), jnp.float32),
            pltpu.VMEM_SHARED((NP,), jnp.float32),
        ],
    )
    def k_deg(dst_hbm, degp_hbm, dstv, onesv, zb, degacc):
        wid = _wid()
        scid = lax.axis_index("c")
        sid = lax.axis_index("s")
        sbase = sid * RPS
        for i in range(CHUNK // LANES):
            onesv[pl.ds(i * LANES, LANES)] = jnp.ones((LANES,), jnp.float32)
        for i in range(RPS // LANES):
            zb[pl.ds(i * LANES, LANES)] = _zeros16()

        def per_g(g, _):
            pltpu.sync_copy(zb, degacc.at[pl.ds(sbase, RPS)])
            plsc.subcore_barrier()
            pltpu.sync_copy(dst_hbm.at[g, wid], dstv)

            def ch(j, _2):
                pltpu.sync_copy(onesv, degacc.at[dstv.at[j]], add=True)
                return _2
            lax.fori_loop(0, C, ch, None)
            plsc.subcore_barrier()
            pltpu.sync_copy(degacc.at[pl.ds(sbase, RPS)],
                            degp_hbm.at[g, scid, pl.ds(sbase, RPS)])
            plsc.subcore_barrier()
            return _
        lax.fori_loop(0, G, per_g, None)

    @functools.partial(
        pl.kernel,
        out_type=(
            jax.ShapeDtypeStruct((G, NP), jnp.float32),
            jax.ShapeDtypeStruct((G, NP, D), jnp.float32),
            jax.ShapeDtypeStruct((G, NP, D), jnp.float32),
        ),
        mesh=mesh,
        compiler_params=pltpu.CompilerParams(use_tc_tiling_on_sc=False),
        scratch_types=[
            pltpu.VMEM((RPT,), jnp.float32),
            pltpu.VMEM((RPT,), jnp.float32),
            pltpu.VMEM((64, D), jnp.float32),
            pltpu.VMEM((64, D), jnp.float32),
            pltpu.VMEM((64, D), jnp.float32),
        ],
    )
    def k_dis(degp_hbm, tbl_hbm, dis_hbm, y_hbm, out_hbm, pbuf, dacc, rb, ob, yb):
        wid = _wid()
        rs = wid * RPT

        def per_g(g, _):
            for i in range(RPT // LANES):
                dacc[pl.ds(i * LANES, LANES)] = _zeros16()

            for u in range(nc):
                pltpu.sync_copy(degp_hbm.at[g, u, pl.ds(rs, RPT)], pbuf)
                for i in range(RPT // LANES):
                    s = pl.ds(i * LANES, LANES)
                    dacc[s] = dacc[s] + pbuf[s]

            for i in range(RPT // LANES):
                s = pl.ds(i * LANES, LANES)
                d = dacc[s]
                r = _rsqrt16(jnp.maximum(d, 1.0))
                dacc[s] = jnp.where(d > 0.0, r, 0.0)
            pltpu.sync_copy(dacc, dis_hbm.at[g, pl.ds(rs, RPT)])

            def per_c(c, _2):
                sl = pl.ds(rs + c * 64, 64)
                pltpu.sync_copy(tbl_hbm.at[g, sl], rb)

                def grp(rg, _3):
                    dvec = dacc[pl.ds(pl.multiple_of(c * 64 + rg * LANES, LANES), LANES)]
                    for r in range(LANES):
                        s = dvec[r]
                        row = rg * LANES + r
                        for k in range(D // LANES):
                            ks = pl.ds(k * LANES, LANES)
                            v = rb[row, ks]
                            ob[row, ks] = v * ALPHA
                            yb[row, ks] = v * s
                    return _3
                lax.fori_loop(0, 64 // LANES, grp, None)
                pltpu.sync_copy(ob, out_hbm.at[g, sl])
                pltpu.sync_copy(yb, y_hbm.at[g, sl])
                return _2
            lax.fori_loop(0, RPT // 64, per_c, None)
            return _
        lax.fori_loop(0, G, per_g, None)

    @functools.partial(
        pl.kernel,
        out_type=jax.ShapeDtypeStruct((G, nc, NP, D), jnp.float32),
        mesh=mesh,
        compiler_params=pltpu.CompilerParams(use_tc_tiling_on_sc=False),
        scratch_types=[
            pltpu.VMEM_SHARED((NP, D), jnp.float32),
            pltpu.VMEM((1, CHUNK), jnp.int32),
            pltpu.VMEM((1, CHUNK), jnp.int32),
            pltpu.VMEM((CHUNK, D), jnp.float32),
            pltpu.VMEM((64, D), jnp.float32),
            pltpu.SemaphoreType.DMA,
        ],
    )
    def k_scat(src_hbm, dst_hbm, y_hbm, p_hbm, acc, sidx, didx, rows, zbuf, sem):
        wid = _wid()
        scid = lax.axis_index("c")
        sid = lax.axis_index("s")
        sbase = sid * RPS

        def zrow(r, _):
            for k in range(D // LANES):
                zbuf[r, pl.ds(k * LANES, LANES)] = _zeros16()
            return _
        lax.fori_loop(0, 64, zrow, None)

        def per_g(g, _):
            def zb(b, _2):
                pltpu.sync_copy(zbuf, acc.at[pl.ds(sbase + b * 64, 64)])
                return _2
            lax.fori_loop(0, RPS // 64, zb, None)
            plsc.subcore_barrier()

            def ch(j, _2):
                pltpu.sync_copy(src_hbm.at[g, wid, j], sidx.at[0])
                pltpu.sync_copy(dst_hbm.at[g, wid, j], didx.at[0])
                pltpu.async_copy(y_hbm.at[g].at[sidx.at[0]], rows, sem).wait()
                pltpu.sync_copy(rows, acc.at[didx.at[0]], add=True)
                return _2
            lax.fori_loop(0, C, ch, None)
            plsc.subcore_barrier()

            pltpu.sync_copy(acc.at[pl.ds(sbase, RPS)],
                            p_hbm.at[g, scid, pl.ds(sbase, RPS)])
            plsc.subcore_barrier()
            return _
        lax.fori_loop(0, G, per_g, None)

    @functools.partial(
        pl.kernel,
        out_type=(
            jax.ShapeDtypeStruct((G, NP, D), jnp.float32),
            jax.ShapeDtypeStruct((G, NP, D), jnp.float32),
        ),
        mesh=mesh,
        compiler_params=pltpu.CompilerParams(use_tc_tiling_on_sc=False),
        scratch_types=[
            pltpu.VMEM((RPT,), jnp.float32),
            pltpu.VMEM((64, D), jnp.float32),
            pltpu.VMEM((64, D), jnp.float32),
            pltpu.VMEM((64, D), jnp.float32),
            pltpu.VMEM((64, D), jnp.float32),
        ],
    )
    def k_merge(p_hbm, dis_hbm, outp_hbm, outn_hbm, yn_hbm, disb, p0b, p1b, ob, yb):
        wid = _wid()
        rs = wid * RPT

        def per_g(g, _):
            pltpu.sync_copy(dis_hbm.at[g, pl.ds(rs, RPT)], disb)

            def per_c(c, _2):
                sl = pl.ds(rs + c * 64, 64)
                pltpu.sync_copy(p_hbm.at[g, 0, sl], p0b)
                if nc > 1:
                    pltpu.sync_copy(p_hbm.at[g, 1, sl], p1b)
                pltpu.sync_copy(outp_hbm.at[g, sl], ob)

                def grp(rg, _3):
                    dvec = disb[pl.ds(pl.multiple_of(c * 64 + rg * LANES, LANES), LANES)]
                    for r in range(LANES):
                        s = dvec[r]
                        row = rg * LANES + r
                        for k in range(D // LANES):
                            ks = pl.ds(k * LANES, LANES)
                            v = p0b[row, ks]
                            if nc > 1:
                                v = v + p1b[row, ks]
                            x = v * s
                            ob[row, ks] = ob[row, ks] + x * ALPHA
                            yb[row, ks] = x * s
                    return _3
                lax.fori_loop(0, 64 // LANES, grp, None)
                pltpu.sync_copy(ob, outn_hbm.at[g, sl])
                pltpu.sync_copy(yb, yn_hbm.at[g, sl])
                return _2
            lax.fori_loop(0, RPT // 64, per_c, None)
            return _
        lax.fori_loop(0, G, per_g, None)

    @functools.partial(
        pl.kernel,
        out_type=jax.ShapeDtypeStruct((G, T, C, CHUNK), jnp.float32),
        mesh=mesh,
        compiler_params=pltpu.CompilerParams(use_tc_tiling_on_sc=False),
        scratch_types=[
            pltpu.VMEM((C, CHUNK), jnp.int32),
            pltpu.VMEM((C, CHUNK), jnp.int32),
            pltpu.VMEM((CHUNK, D), jnp.float32),
            pltpu.VMEM((CHUNK, D), jnp.float32),
            pltpu.VMEM((CHUNK,), jnp.float32),
            pltpu.SemaphoreType.DMA,
            pltpu.SemaphoreType.DMA,
        ],
    )
    def k_rank(src_hbm, dst_hbm, o_hbm, r_hbm, srcv, dstv, ab, bb, rbuf, sa, sb):
        wid = _wid()

        def per_g(g, _):
            pltpu.sync_copy(src_hbm.at[g, wid], srcv)
            pltpu.sync_copy(dst_hbm.at[g, wid], dstv)

            def ch(j, _2):
                ca = pltpu.async_copy(o_hbm.at[g].at[srcv.at[j]], ab, sa)
                cb = pltpu.async_copy(o_hbm.at[g].at[dstv.at[j]], bb, sb)
                ca.wait()
                cb.wait()
                lanes = lax.iota(jnp.int32, LANES)

                def grp(rg, _3):
                    rvec = _zeros16()
                    for r in range(LANES):
                        row = rg * LANES + r
                        acc = ab[row, pl.ds(0, LANES)] * bb[row, pl.ds(0, LANES)]
                        for k in range(1, D // LANES):
                            ks = pl.ds(k * LANES, LANES)
                            acc = acc + ab[row, ks] * bb[row, ks]
                        for sh in (8, 4, 2, 1):
                            acc = acc + acc.at[lanes ^ sh].get(
                                mode="promise_in_bounds")
                        rvec = jnp.where(lanes == r, acc, rvec)
                    rbuf[pl.ds(pl.multiple_of(rg * LANES, LANES), LANES)] = rvec
                    return _3
                lax.fori_loop(0, CHUNK // LANES, grp, None)
                pltpu.sync_copy(rbuf, r_hbm.at[g, wid, j])
                return _2
            lax.fori_loop(0, C, ch, None)
            return _
        lax.fori_loop(0, G, per_g, None)

    return k_deg, k_dis, k_scat, k_merge, k_rank, T, C, EP, NP


def kernel(user_item_edge_index, user_test_edge_index, user_tag_edge_index,
           user_item_table, user_test_table, user_tag_table):
    info = plsc.get_sparse_core_info()
    nc, ns = info.num_cores, info.num_subcores
    k_deg, k_dis, k_scat, k_merge, k_rank, T, C, EP, NP = _build(nc, ns)

    tbl = jnp.stack([user_item_table, user_test_table, user_tag_table])
    tbl = jnp.pad(tbl, ((0, 0), (0, NP - N), (0, 0)))

    def prep(ei):
        pad = jnp.full((2, EP - E), NP - 1, jnp.int32)
        return jnp.concatenate([ei, pad], axis=1).reshape(2, T, C, CHUNK)

    es = jnp.stack([prep(e) for e in (user_item_edge_index,
                                      user_test_edge_index,
                                      user_tag_edge_index)])
    src = es[:, 0]
    dst = es[:, 1]

    degp = k_deg(dst)
    dis, y, out = k_dis(degp, tbl)
    for _ in range(L):
        p = k_scat(src, dst, y)
        out, y = k_merge(p, dis, out)
    r = k_rank(src, dst, out)
    return r.reshape(G, EP)[:, :E]


# speedup vs baseline: 5.0071x; 1.2452x over previous
import functools

import jax
import jax.numpy as jnp
from jax import lax
from jax.experimental import pallas as pl
from jax.experimental.pallas import tpu as pltpu
from jax.experimental.pallas import tpu_sc as plsc

N = 10000
D = 128
E = 320000
G = 3
L = 3
ALPHA = 0.25
LANES = 16
CHUNK = 128


def _zeros16():
    return jnp.zeros((LANES,), jnp.float32)


def _rsqrt16(d):
    i = lax.bitcast_convert_type(d, jnp.int32)
    i = jnp.full((LANES,), 0x5F3759DF, jnp.int32) - lax.shift_right_arithmetic(
        i, jnp.full((LANES,), 1, jnp.int32))
    r = lax.bitcast_convert_type(i, jnp.float32)
    for _ in range(3):
        r = r * (1.5 - 0.5 * d * r * r)
    return r


def _build(nc, ns):
    T = nc * ns
    C = -(-E // (T * CHUNK))
    EP = T * C * CHUNK
    NP = ((N + T * LANES - 1) // (T * LANES)) * (T * LANES)
    RPT = NP // T
    RPS = NP // ns
    mesh = plsc.VectorSubcoreMesh(core_axis_name="c", subcore_axis_name="s")

    def _wid():
        return lax.axis_index("s") * nc + lax.axis_index("c")

    @functools.partial(
        pl.kernel,
        out_type=jax.ShapeDtypeStruct((G, nc, NP), jnp.float32),
        mesh=mesh,
        compiler_params=pltpu.CompilerParams(use_tc_tiling_on_sc=False),
        scratch_types=[
            pltpu.VMEM((1, CHUNK), jnp.int32),
            pltpu.VMEM((1, CHUNK), jnp.int32),
            pltpu.VMEM((CHUNK,), jnp.float32),
            pltpu.VMEM((RPS,), jnp.float32),
            pltpu.VMEM_SHARED((NP,), jnp.float32),
            pltpu.SemaphoreType.DMA,
            pltpu.SemaphoreType.DMA,
        ],
    )
    def k_deg(dst_hbm, degp_hbm, dxa, dxb, onesv, zb, degacc, sa, sb):
        wid = _wid()
        scid = lax.axis_index("c")
        sid = lax.axis_index("s")
        sbase = sid * RPS
        for i in range(CHUNK // LANES):
            onesv[pl.ds(i * LANES, LANES)] = jnp.ones((LANES,), jnp.float32)
        for i in range(RPS // LANES):
            zb[pl.ds(i * LANES, LANES)] = _zeros16()

        def per_g(g, _):
            pltpu.sync_copy(zb, degacc.at[pl.ds(sbase, RPS)])
            plsc.subcore_barrier()

            pltpu.async_copy(dst_hbm.at[g, wid, 0], dxa.at[0], sa)

            def pair(p, _2):
                j0 = 2 * p
                pltpu.async_copy(dst_hbm.at[g, wid, j0 + 1], dxb.at[0], sb)
                pltpu.make_async_copy(dst_hbm.at[g, wid, j0], dxa.at[0], sa).wait()
                pltpu.sync_copy(onesv, degacc.at[dxa.at[0]], add=True)
                pltpu.async_copy(dst_hbm.at[g, wid, j0 + 2], dxa.at[0], sa)
                pltpu.make_async_copy(dst_hbm.at[g, wid, j0], dxb.at[0], sb).wait()
                pltpu.sync_copy(onesv, degacc.at[dxb.at[0]], add=True)
                return _2
            lax.fori_loop(0, (C - 1) // 2, pair, None)
            pltpu.make_async_copy(dst_hbm.at[g, wid, 0], dxa.at[0], sa).wait()
            pltpu.sync_copy(onesv, degacc.at[dxa.at[0]], add=True)

            plsc.subcore_barrier()
            pltpu.sync_copy(degacc.at[pl.ds(sbase, RPS)],
                            degp_hbm.at[g, scid, pl.ds(sbase, RPS)])
            plsc.subcore_barrier()
            return _
        lax.fori_loop(0, G, per_g, None)

    @functools.partial(
        pl.kernel,
        out_type=(
            jax.ShapeDtypeStruct((G, NP), jnp.float32),
            jax.ShapeDtypeStruct((G, NP, D), jnp.float32),
            jax.ShapeDtypeStruct((G, NP, D), jnp.float32),
        ),
        mesh=mesh,
        compiler_params=pltpu.CompilerParams(use_tc_tiling_on_sc=False),
        scratch_types=[
            pltpu.VMEM((RPT,), jnp.float32),
            pltpu.VMEM((RPT,), jnp.float32),
            pltpu.VMEM((64, D), jnp.float32),
            pltpu.VMEM((64, D), jnp.float32),
            pltpu.VMEM((64, D), jnp.float32),
        ],
    )
    def k_dis(degp_hbm, tbl_hbm, dis_hbm, y_hbm, out_hbm, pbuf, dacc, rb, ob, yb):
        wid = _wid()
        rs = wid * RPT

        def per_g(g, _):
            for i in range(RPT // LANES):
                dacc[pl.ds(i * LANES, LANES)] = _zeros16()

            for u in range(nc):
                pltpu.sync_copy(degp_hbm.at[g, u, pl.ds(rs, RPT)], pbuf)
                for i in range(RPT // LANES):
                    s = pl.ds(i * LANES, LANES)
                    dacc[s] = dacc[s] + pbuf[s]

            for i in range(RPT // LANES):
                s = pl.ds(i * LANES, LANES)
                d = dacc[s]
                r = _rsqrt16(jnp.maximum(d, 1.0))
                dacc[s] = jnp.where(d > 0.0, r, 0.0)
            pltpu.sync_copy(dacc, dis_hbm.at[g, pl.ds(rs, RPT)])

            def per_c(c, _2):
                sl = pl.ds(rs + c * 64, 64)
                pltpu.sync_copy(tbl_hbm.at[g, sl], rb)

                def grp(rg, _3):
                    dvec = dacc[pl.ds(pl.multiple_of(c * 64 + rg * LANES, LANES), LANES)]
                    for r in range(LANES):
                        s = dvec[r]
                        row = rg * LANES + r
                        for k in range(D // LANES):
                            ks = pl.ds(k * LANES, LANES)
                            v = rb[row, ks]
                            ob[row, ks] = v * ALPHA
                            yb[row, ks] = v * s
                    return _3
                lax.fori_loop(0, 64 // LANES, grp, None)
                pltpu.sync_copy(ob, out_hbm.at[g, sl])
                pltpu.sync_copy(yb, y_hbm.at[g, sl])
                return _2
            lax.fori_loop(0, RPT // 64, per_c, None)
            return _
        lax.fori_loop(0, G, per_g, None)

    @functools.partial(
        pl.kernel,
        out_type=jax.ShapeDtypeStruct((G, nc, NP, D), jnp.float32),
        mesh=mesh,
        compiler_params=pltpu.CompilerParams(use_tc_tiling_on_sc=False),
        scratch_types=[
            pltpu.VMEM_SHARED((NP, D), jnp.float32),
            pltpu.VMEM((1, CHUNK), jnp.int32),
            pltpu.VMEM((1, CHUNK), jnp.int32),
            pltpu.VMEM((1, CHUNK), jnp.int32),
            pltpu.VMEM((CHUNK, D), jnp.float32),
            pltpu.VMEM((CHUNK, D), jnp.float32),
            pltpu.VMEM((16, D), jnp.float32),
            pltpu.SemaphoreType.DMA,
            pltpu.SemaphoreType.DMA,
        ],
    )
    def k_scat(src_hbm, dst_hbm, y_hbm, p_hbm, acc, sxa, sxb, didx, rowsa, rowsb,
               zbuf, sa, sb):
        wid = _wid()
        scid = lax.axis_index("c")
        sid = lax.axis_index("s")
        sbase = sid * RPS

        def zrow(r, _):
            for k in range(D // LANES):
                zbuf[r, pl.ds(k * LANES, LANES)] = _zeros16()
            return _
        lax.fori_loop(0, 16, zrow, None)

        def per_g(g, _):
            def zb(b, _2):
                pltpu.sync_copy(zbuf, acc.at[pl.ds(sbase + b * 16, 16)])
                return _2
            lax.fori_loop(0, RPS // 16, zb, None)
            plsc.subcore_barrier()

            pltpu.sync_copy(src_hbm.at[g, wid, 0], sxa.at[0])
            pltpu.async_copy(y_hbm.at[g].at[sxa.at[0]], rowsa, sa)

            def pair(p, _2):
                j0 = 2 * p
                pltpu.sync_copy(src_hbm.at[g, wid, j0 + 1], sxb.at[0])
                pltpu.async_copy(y_hbm.at[g].at[sxb.at[0]], rowsb, sb)
                pltpu.sync_copy(dst_hbm.at[g, wid, j0], didx.at[0])
                pltpu.make_async_copy(y_hbm.at[g].at[sxa.at[0]], rowsa, sa).wait()
                pltpu.sync_copy(rowsa, acc.at[didx.at[0]], add=True)
                pltpu.sync_copy(src_hbm.at[g, wid, j0 + 2], sxa.at[0])
                pltpu.async_copy(y_hbm.at[g].at[sxa.at[0]], rowsa, sa)
                pltpu.sync_copy(dst_hbm.at[g, wid, j0 + 1], didx.at[0])
                pltpu.make_async_copy(y_hbm.at[g].at[sxb.at[0]], rowsb, sb).wait()
                pltpu.sync_copy(rowsb, acc.at[didx.at[0]], add=True)
                return _2
            lax.fori_loop(0, (C - 1) // 2, pair, None)
            pltpu.sync_copy(dst_hbm.at[g, wid, C - 1], didx.at[0])
            pltpu.make_async_copy(y_hbm.at[g].at[sxa.at[0]], rowsa, sa).wait()
            pltpu.sync_copy(rowsa, acc.at[didx.at[0]], add=True)
            plsc.subcore_barrier()

            pltpu.sync_copy(acc.at[pl.ds(sbase, RPS)],
                            p_hbm.at[g, scid, pl.ds(sbase, RPS)])
            plsc.subcore_barrier()
            return _
        lax.fori_loop(0, G, per_g, None)

    @functools.partial(
        pl.kernel,
        out_type=(
            jax.ShapeDtypeStruct((G, NP, D), jnp.float32),
            jax.ShapeDtypeStruct((G, NP, D), jnp.float32),
        ),
        mesh=mesh,
        compiler_params=pltpu.CompilerParams(use_tc_tiling_on_sc=False),
        scratch_types=[
            pltpu.VMEM((RPT,), jnp.float32),
            pltpu.VMEM((64, D), jnp.float32),
            pltpu.VMEM((64, D), jnp.float32),
            pltpu.VMEM((64, D), jnp.float32),
            pltpu.VMEM((64, D), jnp.float32),
        ],
    )
    def k_merge(p_hbm, dis_hbm, outp_hbm, outn_hbm, yn_hbm, disb, p0b, p1b, ob, yb):
        wid = _wid()
        rs = wid * RPT

        def per_g(g, _):
            pltpu.sync_copy(dis_hbm.at[g, pl.ds(rs, RPT)], disb)

            def per_c(c, _2):
                sl = pl.ds(rs + c * 64, 64)
                pltpu.sync_copy(p_hbm.at[g, 0, sl], p0b)
                if nc > 1:
                    pltpu.sync_copy(p_hbm.at[g, 1, sl], p1b)
                pltpu.sync_copy(outp_hbm.at[g, sl], ob)

                def grp(rg, _3):
                    dvec = disb[pl.ds(pl.multiple_of(c * 64 + rg * LANES, LANES), LANES)]
                    for r in range(LANES):
                        s = dvec[r]
                        row = rg * LANES + r
                        for k in range(D // LANES):
                            ks = pl.ds(k * LANES, LANES)
                            v = p0b[row, ks]
                            if nc > 1:
                                v = v + p1b[row, ks]
                            x = v * s
                            ob[row, ks] = ob[row, ks] + x * ALPHA
                            yb[row, ks] = x * s
                    return _3
                lax.fori_loop(0, 64 // LANES, grp, None)
                pltpu.sync_copy(ob, outn_hbm.at[g, sl])
                pltpu.sync_copy(yb, yn_hbm.at[g, sl])
                return _2
            lax.fori_loop(0, RPT // 64, per_c, None)
            return _
        lax.fori_loop(0, G, per_g, None)

    @functools.partial(
        pl.kernel,
        out_type=jax.ShapeDtypeStruct((G, T, C, CHUNK), jnp.float32),
        mesh=mesh,
        compiler_params=pltpu.CompilerParams(use_tc_tiling_on_sc=False),
        scratch_types=[
            pltpu.VMEM((2, 1, CHUNK), jnp.int32),
            pltpu.VMEM((2, 1, CHUNK), jnp.int32),
            pltpu.VMEM((2, CHUNK, D), jnp.float32),
            pltpu.VMEM((2, CHUNK, D), jnp.float32),
            pltpu.VMEM((CHUNK,), jnp.float32),
            pltpu.SemaphoreType.DMA,
            pltpu.SemaphoreType.DMA,
        ],
    )
    def k_rank(src_hbm, dst_hbm, o_hbm, r_hbm, sx, dx, ab, bb, rbuf, sa, sb):
        wid = _wid()

        def fire(g, j, h, sem):
            pltpu.sync_copy(src_hbm.at[g, wid, j], sx.at[h, 0])
            pltpu.sync_copy(dst_hbm.at[g, wid, j], dx.at[h, 0])
            pltpu.async_copy(o_hbm.at[g].at[sx.at[h, 0]], ab.at[h], sem)
            pltpu.async_copy(o_hbm.at[g].at[dx.at[h, 0]], bb.at[h], sem)

        def finish(g, j, h, sem):
            pltpu.make_async_copy(o_hbm.at[g].at[sx.at[h, 0]], ab.at[h], sem).wait()
            pltpu.make_async_copy(o_hbm.at[g].at[dx.at[h, 0]], bb.at[h], sem).wait()
            lanes = lax.iota(jnp.int32, LANES)

            def grp(rg, _3):
                rvec = _zeros16()
                for r in range(LANES):
                    row = rg * LANES + r
                    acc = ab[h, row, pl.ds(0, LANES)] * bb[h, row, pl.ds(0, LANES)]
                    for k in range(1, D // LANES):
                        ks = pl.ds(k * LANES, LANES)
                        acc = acc + ab[h, row, ks] * bb[h, row, ks]
                    for sh in (8, 4, 2, 1):
                        acc = acc + acc.at[lanes ^ sh].get(
                            mode="promise_in_bounds")
                    rvec = jnp.where(lanes == r, acc, rvec)
                rbuf[pl.ds(pl.multiple_of(rg * LANES, LANES), LANES)] = rvec
                return _3
            lax.fori_loop(0, CHUNK // LANES, grp, None)
            pltpu.sync_copy(rbuf, r_hbm.at[g, wid, j])

        def per_g(g, _):
            fire(g, 0, 0, sa)

            def pair(p, _2):
                j0 = 2 * p
                fire(g, j0 + 1, 1, sb)
                finish(g, j0, 0, sa)
                fire(g, j0 + 2, 0, sa)
                finish(g, j0 + 1, 1, sb)
                return _2
            lax.fori_loop(0, (C - 1) // 2, pair, None)
            finish(g, C - 1, 0, sa)
            return _
        lax.fori_loop(0, G, per_g, None)

    return k_deg, k_dis, k_scat, k_merge, k_rank, T, C, EP, NP


def kernel(user_item_edge_index, user_test_edge_index, user_tag_edge_index,
           user_item_table, user_test_table, user_tag_table):
    info = plsc.get_sparse_core_info()
    nc, ns = info.num_cores, info.num_subcores
    k_deg, k_dis, k_scat, k_merge, k_rank, T, C, EP, NP = _build(nc, ns)

    tbl = jnp.stack([user_item_table, user_test_table, user_tag_table])
    tbl = jnp.pad(tbl, ((0, 0), (0, NP - N), (0, 0)))

    def prep(ei):
        pad = jnp.full((2, EP - E), NP - 1, jnp.int32)
        return jnp.concatenate([ei, pad], axis=1).reshape(2, T, C, CHUNK)

    es = jnp.stack([prep(e) for e in (user_item_edge_index,
                                      user_test_edge_index,
                                      user_tag_edge_index)])
    src = es[:, 0]
    dst = es[:, 1]

    degp = k_deg(dst)
    dis, y, out = k_dis(degp, tbl)
    for _ in range(L):
        p = k_scat(src, dst, y)
        out, y = k_merge(p, dis, out)
    r = k_rank(src, dst, out)
    return r.reshape(G, EP)[:, :E]
